# baseline (speedup 1.0000x reference)
import numpy as np
import jax
import jax.numpy as jnp
from jax import lax
from jax.experimental import pallas as pl
from jax.experimental.pallas import tpu as pltpu
from jax.experimental.pallas import tpu_sc as plsc

_B = 16
_S = 512
_P = _B * _S
_NV = 8
_D = 256
_NMOD = 17
_ROWS = _P * _NMOD
_TROWS = _P * 9

_NW = 32
_PPW = _P // _NW
_GP = 8
_CR2 = _GP * _NMOD * 2
_SR2 = _GP * 9 * 2
_MT2 = _SR2
_NCH = _PPW // _GP


def _pos_encoding_np(seq_len, d_model):
    pos = np.arange(seq_len, dtype=np.float32)[:, None]
    div = np.exp(np.arange(0, d_model, 2, dtype=np.float32) * (-np.log(10000.0) / d_model))
    pe = np.zeros((seq_len, d_model), dtype=np.float32)
    pe[:, 0::2] = np.sin(pos * div)
    pe[:, 1::2] = np.cos(pos * div)
    return pe


_PE2 = _pos_encoding_np(_S, _D).reshape(_S * 2, 128)


def _sc_body(tbf, ridx, pe, emb, out,
             gbuf, obuf, pev, embv, ridxc, gsem, psem, osem):
    wid = lax.axis_index("s") * 2 + lax.axis_index("c")
    pair0 = wid * _PPW

    pltpu.sync_copy(emb, embv)
    pltpu.sync_copy(tbf.at[pl.ds(_TROWS * 2, 8)],
                    gbuf.at[0, pl.ds(_SR2, 8)])
    pltpu.sync_copy(tbf.at[pl.ds(_TROWS * 2, 8)],
                    gbuf.at[1, pl.ds(_SR2, 8)])

    def fire_in(k, buf):
        p0 = pair0 + k * _GP
        s0 = lax.rem(p0, _S)
        pltpu.async_copy(tbf.at[pl.ds(p0 * 18, 72)],
                         gbuf.at[buf, pl.ds(0, 72)], gsem)
        pltpu.async_copy(tbf.at[pl.ds(p0 * 18 + 72, 72)],
                         gbuf.at[buf, pl.ds(72, 72)], gsem)
        pltpu.async_copy(pe.at[pl.ds(s0 * 2, _GP * 2)], pev.at[buf], psem)
        pltpu.async_copy(ridx.at[pl.ds(p0, _GP)], ridxc.at[buf], psem)

    def drain(sem, dst):
        pltpu.make_async_copy(tbf.at[pl.ds(0, dst.shape[0])], dst, sem).wait()

    fire_in(0, 0)

    def chunk(k, _):
        b = lax.rem(k, 2)
        nb = 1 - b
        p0 = pair0 + k * _GP

        @pl.when(k + 1 < _NCH)
        def _():
            fire_in(k + 1, nb)

        drain(gsem, gbuf.at[b, pl.ds(0, 72)])
        drain(gsem, gbuf.at[b, pl.ds(72, 72)])
        drain(psem, pev.at[b])
        drain(psem, ridxc.at[b])

        @pl.when(k >= 2)
        def _():
            drain(osem, obuf.at[b, pl.ds(0, 136)])
            drain(osem, obuf.at[b, pl.ds(136, 136)])

        @plsc.parallel_loop(0, _GP, unroll=1)
        def _j_loop(j):
            pec = [pev[b, 2 * j + c // 8, pl.ds((c % 8) * 16, 16)]
                   for c in range(16)]
            vvec = ridxc[b, j, pl.ds(0, 16)]
            grows = jnp.where(vvec < _NV, (j * 9 + 1 + vvec) * 2, _MT2)
            vals = [gbuf[b, j * 18 + c // 8, pl.ds((c % 8) * 16, 16)]
                    + embv[c // 8, pl.ds((c % 8) * 16, 16)] + pec[c]
                    for c in range(16)]
            for c in range(16):
                obuf[b, j * _NMOD * 2 + c // 8, pl.ds((c % 8) * 16, 16)] = vals[c]

            for m in range(1, _NMOD):
                grow = grows[m - 1]
                orow = (j * _NMOD + m) * 2
                vals = [gbuf[b, grow + c // 8, pl.ds((c % 8) * 16, 16)]
                        + embv[2 * m + c // 8, pl.ds((c % 8) * 16, 16)]
                        + pec[c]
                        for c in range(16)]
                for c in range(16):
                    obuf[b, orow + c // 8, pl.ds((c % 8) * 16, 16)] = vals[c]

        pltpu.async_copy(obuf.at[b, pl.ds(0, 136)],
                         out.at[pl.ds(p0 * _NMOD * 2, 136)], osem)
        pltpu.async_copy(obuf.at[b, pl.ds(136, 136)],
                         out.at[pl.ds(p0 * _NMOD * 2 + 136, 136)], osem)
        return _

    lax.fori_loop(0, _NCH, chunk, None)

    drain(osem, obuf.at[0, pl.ds(0, 136)])
    drain(osem, obuf.at[0, pl.ds(0, 136)])
    drain(osem, obuf.at[0, pl.ds(0, 136)])
    drain(osem, obuf.at[0, pl.ds(0, 136)])


_revert_sc = pl.kernel(
    _sc_body,
    out_type=jax.ShapeDtypeStruct((_ROWS * 2, 128), jnp.float32),
    mesh=plsc.VectorSubcoreMesh(core_axis_name="c", subcore_axis_name="s"),
    scratch_types=[
        pltpu.VMEM((2, _SR2 + 8, 128), jnp.float32),
        pltpu.VMEM((2, _CR2, 128), jnp.float32),
        pltpu.VMEM((2, _GP * 2, 128), jnp.float32),
        pltpu.VMEM((_NMOD * 2, 128), jnp.float32),
        pltpu.VMEM((2, _GP, 128), jnp.int32),
        pltpu.SemaphoreType.DMA,
        pltpu.SemaphoreType.DMA,
        pltpu.SemaphoreType.DMA,
    ],
)


def kernel(temporal_block, temporal_masked_idx, temporal_revert_idx,
           mask_token_param, temporal_mod_emb_table):
    del temporal_masked_idx
    tbf = jnp.concatenate(
        [temporal_block.reshape(_TROWS, _D), mask_token_param.reshape(1, _D),
         jnp.zeros((3, _D), jnp.float32)],
        axis=0).reshape(_TROWS * 2 + 8, 128)
    ridxp = jnp.pad(
        temporal_revert_idx.reshape(_P, 16).astype(jnp.int32),
        ((0, 0), (0, 112)))
    pe = jnp.asarray(_PE2)
    emb2 = temporal_mod_emb_table.reshape(_NMOD * 2, 128)
    out = _revert_sc(tbf, ridxp, pe, emb2)
    return out.reshape(_B, _S, _NMOD, _D)


# speedup vs baseline: 1.3393x; 1.3393x over previous
import numpy as np
import jax
import jax.numpy as jnp
from jax import lax
from jax.experimental import pallas as pl
from jax.experimental.pallas import tpu as pltpu
from jax.experimental.pallas import tpu_sc as plsc

_B = 16
_S = 512
_P = _B * _S
_NV = 8
_D = 256
_NMOD = 17
_ROWS = _P * _NMOD
_TROWS = _P * 9
_MASKROW = _TROWS

_NW = 32
_PPW = _P // _NW
_GP = 8
_CR = _GP * _NMOD
_SR = _GP * 9
_MTROW = _SR
_NCH = _PPW // _GP


def _pos_encoding_np(seq_len, d_model):
    pos = np.arange(seq_len, dtype=np.float32)[:, None]
    div = np.exp(np.arange(0, d_model, 2, dtype=np.float32) * (-np.log(10000.0) / d_model))
    pe = np.zeros((seq_len, d_model), dtype=np.float32)
    pe[:, 0::2] = np.sin(pos * div)
    pe[:, 1::2] = np.cos(pos * div)
    return pe


_PE = _pos_encoding_np(_S, _D)


def _sc_body(tbf, ridx, pe, emb, out,
             gbuf, obuf, pev, embv, ridxv, gsem, psem, osem):
    wid = lax.axis_index("s") * 2 + lax.axis_index("c")
    pair0 = wid * _PPW

    pltpu.sync_copy(emb, embv)
    pltpu.sync_copy(ridx.at[pl.ds(pair0 * 16, _PPW * 16)], ridxv)
    pltpu.sync_copy(tbf.at[pl.ds(_MASKROW, 1)], gbuf.at[0, pl.ds(_MTROW, 1)])
    pltpu.sync_copy(tbf.at[pl.ds(_MASKROW, 1)], gbuf.at[1, pl.ds(_MTROW, 1)])

    def fire_in(k, buf):
        p0 = pair0 + k * _GP
        s0 = lax.rem(p0, _S)
        pltpu.async_copy(tbf.at[pl.ds(p0 * 9, 40)],
                         gbuf.at[buf, pl.ds(0, 40)], gsem)
        pltpu.async_copy(tbf.at[pl.ds(p0 * 9 + 40, 32)],
                         gbuf.at[buf, pl.ds(40, 32)], gsem)
        pltpu.async_copy(pe.at[pl.ds(s0, _GP)], pev.at[buf], psem)

    def drain(sem, dst):
        pltpu.make_async_copy(tbf.at[pl.ds(0, dst.shape[0])], dst, sem).wait()

    fire_in(0, 0)

    def chunk(k, _):
        b = lax.rem(k, 2)
        nb = 1 - b
        p0 = pair0 + k * _GP

        @pl.when(k + 1 < _NCH)
        def _():
            fire_in(k + 1, nb)

        drain(gsem, gbuf.at[b, pl.ds(0, 40)])
        drain(gsem, gbuf.at[b, pl.ds(40, 32)])
        drain(psem, pev.at[b])

        @pl.when(k >= 2)
        def _():
            drain(osem, obuf.at[b, pl.ds(0, 72)])
            drain(osem, obuf.at[b, pl.ds(72, 64)])

        @plsc.parallel_loop(0, _GP, unroll=1)
        def _j_loop(j):
            pec = [pev[b, j, pl.ds(c * 16, 16)] for c in range(16)]
            vvec = ridxv[pl.ds((k * _GP + j) * 16, 16)]
            grows = jnp.where(vvec < _NV, j * 9 + 1 + vvec, _MTROW)
            vals = [gbuf[b, j * 9, pl.ds(c * 16, 16)]
                    + embv[0, pl.ds(c * 16, 16)] + pec[c] for c in range(16)]
            for c in range(16):
                obuf[b, j * _NMOD, pl.ds(c * 16, 16)] = vals[c]

            for m in range(1, _NMOD):
                grow = grows[m - 1]
                orow = j * _NMOD + m
                vals = [gbuf[b, grow, pl.ds(c * 16, 16)]
                        + embv[m, pl.ds(c * 16, 16)] + pec[c]
                        for c in range(16)]
                for c in range(16):
                    obuf[b, orow, pl.ds(c * 16, 16)] = vals[c]

        pltpu.async_copy(obuf.at[b, pl.ds(0, 72)],
                         out.at[pl.ds(p0 * _NMOD, 72)], osem)
        pltpu.async_copy(obuf.at[b, pl.ds(72, 64)],
                         out.at[pl.ds(p0 * _NMOD + 72, 64)], osem)
        return _

    lax.fori_loop(0, _NCH, chunk, None)

    drain(osem, obuf.at[0, pl.ds(0, 72)])
    drain(osem, obuf.at[0, pl.ds(72, 64)])
    drain(osem, obuf.at[0, pl.ds(0, 72)])
    drain(osem, obuf.at[0, pl.ds(72, 64)])


_revert_sc = pl.kernel(
    _sc_body,
    out_type=jax.ShapeDtypeStruct((_ROWS, _D), jnp.float32),
    mesh=plsc.VectorSubcoreMesh(core_axis_name="c", subcore_axis_name="s"),
    scratch_types=[
        pltpu.VMEM((2, _SR + 1, _D), jnp.float32),
        pltpu.VMEM((2, _CR, _D), jnp.float32),
        pltpu.VMEM((2, _GP, _D), jnp.float32),
        pltpu.VMEM((_NMOD, _D), jnp.float32),
        pltpu.VMEM((_PPW * 16,), jnp.int32),
        pltpu.SemaphoreType.DMA,
        pltpu.SemaphoreType.DMA,
        pltpu.SemaphoreType.DMA,
    ],
)


def kernel(temporal_block, temporal_masked_idx, temporal_revert_idx,
           mask_token_param, temporal_mod_emb_table):
    del temporal_masked_idx
    tbf = jnp.concatenate(
        [temporal_block.reshape(_TROWS, _D), mask_token_param.reshape(1, _D)],
        axis=0)
    ridxf = temporal_revert_idx.reshape(-1).astype(jnp.int32)
    pe = jnp.asarray(_PE)
    out = _revert_sc(tbf, ridxf, pe, temporal_mod_emb_table)
    return out.reshape(_B, _S, _NMOD, _D)
